# baseline (speedup 1.0000x reference)
import functools

import jax
import jax.numpy as jnp
from jax.experimental import pallas as pl
from jax.experimental.pallas import tpu as pltpu


def _body(t_ref, x_ref, w1_ref, b1_ref, w2_ref, b2_ref, g_ref, bb_ref,
          hw_ref, hb_ref, o_ref, *, n_types):
    x = x_ref[...]
    h = jnp.dot(x, w1_ref[...], preferred_element_type=jnp.float32)
    h = jnp.maximum(h + b1_ref[...], 0.0)
    h = jnp.dot(h, w2_ref[...], preferred_element_type=jnp.float32)
    h = h + b2_ref[...]
    mu = jnp.mean(h, axis=-1, keepdims=True)
    c = h - mu
    var = jnp.mean(c * c, axis=-1, keepdims=True)
    h = c * jax.lax.rsqrt(var + 1e-5) * g_ref[...] + bb_ref[...]
    t = t_ref[...]
    out = None
    for tt in range(n_types):
        mask = t == tt
        ht = jnp.where(mask, h, 0.0)
        y = jnp.dot(ht, hw_ref[tt], preferred_element_type=jnp.float32)
        y = y + jnp.where(mask, hb_ref[tt][None, :], 0.0)
        out = y if out is None else out + y
    o_ref[...] = out


def kernel(node_latent, node_types, w1, b1, w2, b2, ln_gamma, ln_beta,
           head_w, head_b, *, interpret=False):
    n, latent = node_latent.shape
    hidden = w1.shape[1]
    out_d = head_w.shape[2]
    n_types = head_w.shape[0]
    bm = 1000
    grid = (n // bm,)

    t2 = node_types.reshape(n, 1)
    b1r = b1.reshape(1, hidden)
    b2r = b2.reshape(1, hidden)
    gr = ln_gamma.reshape(1, hidden)
    br = ln_beta.reshape(1, hidden)

    return pl.pallas_call(
        functools.partial(_body, n_types=n_types),
        out_shape=jax.ShapeDtypeStruct((n, out_d), jnp.float32),
        grid=grid,
        in_specs=[
            pl.BlockSpec((bm, 1), lambda i: (i, 0)),
            pl.BlockSpec((bm, latent), lambda i: (i, 0)),
            pl.BlockSpec((latent, hidden), lambda i: (0, 0)),
            pl.BlockSpec((1, hidden), lambda i: (0, 0)),
            pl.BlockSpec((hidden, hidden), lambda i: (0, 0)),
            pl.BlockSpec((1, hidden), lambda i: (0, 0)),
            pl.BlockSpec((1, hidden), lambda i: (0, 0)),
            pl.BlockSpec((1, hidden), lambda i: (0, 0)),
            pl.BlockSpec((n_types, hidden, out_d), lambda i: (0, 0, 0)),
            pl.BlockSpec((n_types, out_d), lambda i: (0, 0)),
        ],
        out_specs=pl.BlockSpec((bm, out_d), lambda i: (i, 0)),
        compiler_params=pltpu.CompilerParams(
            dimension_semantics=("parallel",),
            vmem_limit_bytes=56 * 1024 * 1024,
        ),
        name="adaptive_decoder",
        interpret=interpret,
    )(t2, node_latent, w1, b1r, w2, b2r, gr, br, head_w, head_b)


# speedup vs baseline: 1.2426x; 1.2426x over previous
import functools

import jax
import jax.numpy as jnp
from jax.experimental import pallas as pl
from jax.experimental.pallas import tpu as pltpu


def _body(t_ref, x_ref, w1_ref, b1_ref, w2_ref, b2_ref, g_ref, bb_ref,
          hw_ref, hb_ref, o_ref, *, n_types, out_d):
    x = x_ref[...].astype(jnp.bfloat16)
    h = jnp.dot(x, w1_ref[...], preferred_element_type=jnp.float32)
    h = jnp.maximum(h + b1_ref[...], 0.0)
    h = jnp.dot(h.astype(jnp.bfloat16), w2_ref[...],
                preferred_element_type=jnp.float32)
    h = h + b2_ref[...]
    mu = jnp.mean(h, axis=-1, keepdims=True)
    c = h - mu
    var = jnp.mean(c * c, axis=-1, keepdims=True)
    h = c * jax.lax.rsqrt(var + 1e-5) * g_ref[...] + bb_ref[...]
    y = jnp.dot(h.astype(jnp.bfloat16), hw_ref[...],
                preferred_element_type=jnp.float32)
    t = t_ref[...]
    out = None
    for tt in range(n_types):
        sel = jnp.where(t == tt, y[:, tt * out_d:(tt + 1) * out_d]
                        + hb_ref[tt][None, :], 0.0)
        out = sel if out is None else out + sel
    o_ref[...] = out


def kernel(node_latent, node_types, w1, b1, w2, b2, ln_gamma, ln_beta,
           head_w, head_b, *, interpret=False):
    n, latent = node_latent.shape
    hidden = w1.shape[1]
    out_d = head_w.shape[2]
    n_types = head_w.shape[0]
    bm = 1000
    grid = (n // bm,)

    t2 = node_types.reshape(n, 1)
    b1r = b1.reshape(1, hidden)
    b2r = b2.reshape(1, hidden)
    gr = ln_gamma.reshape(1, hidden)
    br = ln_beta.reshape(1, hidden)
    w1b = w1.astype(jnp.bfloat16)
    w2b = w2.astype(jnp.bfloat16)
    hwb = head_w.astype(jnp.bfloat16).transpose(1, 0, 2).reshape(
        hidden, n_types * out_d)

    return pl.pallas_call(
        functools.partial(_body, n_types=n_types, out_d=out_d),
        out_shape=jax.ShapeDtypeStruct((n, out_d), jnp.float32),
        grid=grid,
        in_specs=[
            pl.BlockSpec((bm, 1), lambda i: (i, 0)),
            pl.BlockSpec((bm, latent), lambda i: (i, 0)),
            pl.BlockSpec((latent, hidden), lambda i: (0, 0)),
            pl.BlockSpec((1, hidden), lambda i: (0, 0)),
            pl.BlockSpec((hidden, hidden), lambda i: (0, 0)),
            pl.BlockSpec((1, hidden), lambda i: (0, 0)),
            pl.BlockSpec((1, hidden), lambda i: (0, 0)),
            pl.BlockSpec((1, hidden), lambda i: (0, 0)),
            pl.BlockSpec((hidden, n_types * out_d), lambda i: (0, 0)),
            pl.BlockSpec((n_types, out_d), lambda i: (0, 0)),
        ],
        out_specs=pl.BlockSpec((bm, out_d), lambda i: (i, 0)),
        compiler_params=pltpu.CompilerParams(
            dimension_semantics=("parallel",),
            vmem_limit_bytes=56 * 1024 * 1024,
        ),
        name="adaptive_decoder",
        interpret=interpret,
    )(t2, node_latent, w1b, b1r, w2b, b2r, gr, br, hwb, head_b)


# speedup vs baseline: 1.3944x; 1.1221x over previous
import functools

import jax
import jax.numpy as jnp
from jax.experimental import pallas as pl
from jax.experimental.pallas import tpu as pltpu


def _body(t_ref, x_ref, w1_ref, b1_ref, w2_ref, b2_ref, wp_ref, g1_ref,
          c_ref, o_ref, *, n_types, out_d, hidden):
    x = x_ref[...].astype(jnp.bfloat16)
    h = jnp.dot(x, w1_ref[...], preferred_element_type=jnp.float32)
    h = jnp.maximum(h + b1_ref[...], 0.0)
    h = jnp.dot(h.astype(jnp.bfloat16), w2_ref[...],
                preferred_element_type=jnp.float32)
    h = h + b2_ref[...]
    y = jnp.dot(h.astype(jnp.bfloat16), wp_ref[...],
                preferred_element_type=jnp.float32)
    inv_h = 1.0 / hidden
    mu = jnp.sum(h, axis=-1, keepdims=True) * inv_h
    m2 = jnp.sum(h * h, axis=-1, keepdims=True) * inv_h
    rstd = jax.lax.rsqrt(jnp.maximum(m2 - mu * mu, 0.0) + 1e-5)
    t = t_ref[...]
    y_sel = None
    g_sel = None
    c_sel = None
    for tt in range(n_types):
        mask = t == tt
        sl = slice(tt * out_d, (tt + 1) * out_d)
        ys = jnp.where(mask, y[:, sl], 0.0)
        gs = jnp.where(mask, g1_ref[:, sl], 0.0)
        cs = jnp.where(mask, c_ref[:, sl], 0.0)
        if y_sel is None:
            y_sel, g_sel, c_sel = ys, gs, cs
        else:
            y_sel, g_sel, c_sel = y_sel + ys, g_sel + gs, c_sel + cs
    o_ref[...] = rstd * y_sel - (rstd * mu) * g_sel + c_sel


def kernel(node_latent, node_types, w1, b1, w2, b2, ln_gamma, ln_beta,
           head_w, head_b, *, interpret=False):
    n, latent = node_latent.shape
    hidden = w1.shape[1]
    out_d = head_w.shape[2]
    n_types = head_w.shape[0]
    bm = 1000
    grid = (n // bm,)

    t2 = node_types.reshape(n, 1)
    b1r = b1.reshape(1, hidden)
    b2r = b2.reshape(1, hidden)
    w1b = w1.astype(jnp.bfloat16)
    w2b = w2.astype(jnp.bfloat16)
    w_cat = head_w.transpose(1, 0, 2).reshape(hidden, n_types * out_d)
    wp = (ln_gamma[:, None] * w_cat).astype(jnp.bfloat16)
    g1 = (ln_gamma @ w_cat).reshape(1, n_types * out_d)
    c_all = (ln_beta @ w_cat).reshape(1, n_types * out_d) \
        + head_b.reshape(1, n_types * out_d)

    return pl.pallas_call(
        functools.partial(_body, n_types=n_types, out_d=out_d, hidden=hidden),
        out_shape=jax.ShapeDtypeStruct((n, out_d), jnp.float32),
        grid=grid,
        in_specs=[
            pl.BlockSpec((bm, 1), lambda i: (i, 0)),
            pl.BlockSpec((bm, latent), lambda i: (i, 0)),
            pl.BlockSpec((latent, hidden), lambda i: (0, 0)),
            pl.BlockSpec((1, hidden), lambda i: (0, 0)),
            pl.BlockSpec((hidden, hidden), lambda i: (0, 0)),
            pl.BlockSpec((1, hidden), lambda i: (0, 0)),
            pl.BlockSpec((hidden, n_types * out_d), lambda i: (0, 0)),
            pl.BlockSpec((1, n_types * out_d), lambda i: (0, 0)),
            pl.BlockSpec((1, n_types * out_d), lambda i: (0, 0)),
        ],
        out_specs=pl.BlockSpec((bm, out_d), lambda i: (i, 0)),
        compiler_params=pltpu.CompilerParams(
            dimension_semantics=("parallel",),
            vmem_limit_bytes=56 * 1024 * 1024,
        ),
        name="adaptive_decoder",
        interpret=interpret,
    )(t2, node_latent, w1b, b1r, w2b, b2r, wp, g1, c_all)
